# baseline (speedup 1.0000x reference)
import jax
import jax.numpy as jnp
from jax import lax
from jax.experimental import pallas as pl
from jax.experimental.pallas import tpu as pltpu
from jax.experimental.pallas import tpu_sc as plsc

N = 50000
H = 512
VY = 64

NW = 32
K = 56
NCHS = [2, 8, 10, 8]
C = len(NCHS)
BPWS = [n * K for n in NCHS]
CHUNKS = [bpw * NW for bpw in BPWS]
NPAD = sum(CHUNKS)

BN = 3584
NBCS = [ch // BN for ch in CHUNKS]
BLK_OFFS = [sum(NBCS[:c]) for c in range(C)]
NB = sum(NBCS)


def _make_sc_gather(nch):
    bpw = nch * K

    def body(table_hbm, idx_hbm, out_hbm, idx_v,
             buf0, buf1, buf2, buf3, g0, g1, g2, g3, s0, s1, s2, s3):
        wid = lax.axis_index("s") * 2 + lax.axis_index("c")
        base = wid * bpw
        pltpu.sync_copy(idx_hbm.at[wid], idx_v)
        bufs = (buf0, buf1, buf2, buf3)
        gsems = (g0, g1, g2, g3)
        ssems = (s0, s1, s2, s3)
        gh = [None] * 4
        sh = [None] * 4
        gh[0] = pltpu.async_copy(table_hbm.at[idx_v.at[0]], buf0, g0)
        for j in range(nch):
            p = j % 4
            if j + 1 < nch:
                q = (j + 1) % 4
                if j + 1 >= 4:
                    sh[q].wait()
                gh[q] = pltpu.async_copy(
                    table_hbm.at[idx_v.at[j + 1]], bufs[q], gsems[q])
            gh[p].wait()
            sh[p] = pltpu.async_copy(
                bufs[p], out_hbm.at[pl.ds(base + j * K, K)], ssems[p])
        for p in range(min(nch, 4)):
            sh[(nch - 1 - p) % 4].wait()

    return pl.kernel(
        body,
        out_type=jax.ShapeDtypeStruct((nch * K * NW, H), jnp.float32),
        mesh=plsc.VectorSubcoreMesh(core_axis_name="c", subcore_axis_name="s"),
        scratch_types=[
            pltpu.VMEM((nch, K), jnp.int32),
            pltpu.VMEM((K, H), jnp.float32),
            pltpu.VMEM((K, H), jnp.float32),
            pltpu.VMEM((K, H), jnp.float32),
            pltpu.VMEM((K, H), jnp.float32),
            pltpu.SemaphoreType.DMA,
            pltpu.SemaphoreType.DMA,
            pltpu.SemaphoreType.DMA,
            pltpu.SemaphoreType.DMA,
            pltpu.SemaphoreType.DMA,
            pltpu.SemaphoreType.DMA,
            pltpu.SemaphoreType.DMA,
            pltpu.SemaphoreType.DMA,
        ],
    )


_sc_gathers = [_make_sc_gather(nch) for nch in NCHS]


def _mm_compute(g_ref, ids_ref, wt_ref, tt_ref, wb_ref, b_ref, out_ref, small_ref):
    @pl.when(pl.program_id(0) == 0)
    def _():
        small_ref[...] = (
            jnp.dot(tt_ref[...], wb_ref[...], preferred_element_type=jnp.float32)
            + b_ref[...])

    ids = ids_ref[0, 0, :]
    onehot = (ids[:, None] == lax.broadcasted_iota(jnp.int32, (1, VY), 1)
              ).astype(jnp.float32)
    out_ref[...] = (
        jnp.dot(g_ref[...], wt_ref[...], preferred_element_type=jnp.float32)
        + jnp.dot(onehot, small_ref[...], preferred_element_type=jnp.float32))


def _mm_body_first(g_ref, ids_ref, wt_ref, tt_ref, wb_ref, b_ref, out_ref, small_ref):
    _mm_compute(g_ref, ids_ref, wt_ref, tt_ref, wb_ref, b_ref, out_ref, small_ref)


def _mm_body_chained(g_ref, ids_ref, wt_ref, tt_ref, wb_ref, b_ref, prev_ref,
                     out_ref, small_ref):
    del prev_ref
    _mm_compute(g_ref, ids_ref, wt_ref, tt_ref, wb_ref, b_ref, out_ref, small_ref)


def _make_mm(c):
    off = BLK_OFFS[c]
    in_specs = [
        pl.BlockSpec((BN, H), lambda i: (i, 0)),
        pl.BlockSpec((1, 1, BN), lambda i, off=off: (i + off, 0, 0)),
        pl.BlockSpec((H, H), lambda i: (0, 0)),
        pl.BlockSpec((VY, H), lambda i: (0, 0)),
        pl.BlockSpec((H, H), lambda i: (1, 0)),
        pl.BlockSpec((1, H), lambda i: (0, 0)),
    ]
    kwargs = {}
    body = _mm_body_first
    if c > 0:
        in_specs.append(pl.BlockSpec(memory_space=pl.ANY))
        kwargs["input_output_aliases"] = {6: 0}
        body = _mm_body_chained
    return pl.pallas_call(
        body,
        grid=(NBCS[c],),
        in_specs=in_specs,
        out_specs=pl.BlockSpec((BN, H), lambda i, off=off: (i + off, 0)),
        out_shape=jax.ShapeDtypeStruct((N, H), jnp.float32),
        scratch_shapes=[pltpu.VMEM((VY, H), jnp.float32)],
        **kwargs)


_mms = [_make_mm(c) for c in range(C)]


def kernel(token_ids, type_ids, token_table, type_table, W, b):
    tok = jnp.pad(token_ids.astype(jnp.int32), (0, NPAD - N))
    ty = jnp.pad(type_ids.astype(jnp.int32), (0, NPAD - N))
    ids3d = ty.reshape(NB, 1, BN)
    b2 = b.reshape(1, H)

    gs = []
    row = 0
    for c in range(C):
        idx3d = lax.slice(tok, (row,), (row + CHUNKS[c],)).reshape(NW, NCHS[c], K)
        gs.append(_sc_gathers[c](token_table, idx3d))
        row += CHUNKS[c]
    out = _mms[0](gs[0], ids3d, W, type_table, W, b2)
    for c in range(1, C):
        out = _mms[c](gs[c], ids3d, W, type_table, W, b2, out)
    return out


# speedup vs baseline: 1.0014x; 1.0014x over previous
import jax
import jax.numpy as jnp
from jax import lax
from jax.experimental import pallas as pl
from jax.experimental.pallas import tpu as pltpu
from jax.experimental.pallas import tpu_sc as plsc

N = 50000
H = 512
VY = 64

NW = 32
K = 56
NCHS = [2, 10, 10, 6]
C = len(NCHS)
BPWS = [n * K for n in NCHS]
CHUNKS = [bpw * NW for bpw in BPWS]
NPAD = sum(CHUNKS)

BN = 3584
NBCS = [ch // BN for ch in CHUNKS]
BLK_OFFS = [sum(NBCS[:c]) for c in range(C)]
NB = sum(NBCS)


def _make_sc_gather(nch):
    bpw = nch * K

    def body(table_hbm, idx_hbm, out_hbm, idx_v,
             buf0, buf1, buf2, buf3, g0, g1, g2, g3, s0, s1, s2, s3):
        wid = lax.axis_index("s") * 2 + lax.axis_index("c")
        base = wid * bpw
        pltpu.sync_copy(idx_hbm.at[wid], idx_v)
        bufs = (buf0, buf1, buf2, buf3)
        gsems = (g0, g1, g2, g3)
        ssems = (s0, s1, s2, s3)
        gh = [None] * 4
        sh = [None] * 4
        gh[0] = pltpu.async_copy(table_hbm.at[idx_v.at[0]], buf0, g0)
        for j in range(nch):
            p = j % 4
            if j + 1 < nch:
                q = (j + 1) % 4
                if j + 1 >= 4:
                    sh[q].wait()
                gh[q] = pltpu.async_copy(
                    table_hbm.at[idx_v.at[j + 1]], bufs[q], gsems[q])
            gh[p].wait()
            sh[p] = pltpu.async_copy(
                bufs[p], out_hbm.at[pl.ds(base + j * K, K)], ssems[p])
        for p in range(min(nch, 4)):
            sh[(nch - 1 - p) % 4].wait()

    return pl.kernel(
        body,
        out_type=jax.ShapeDtypeStruct((nch * K * NW, H), jnp.float32),
        mesh=plsc.VectorSubcoreMesh(core_axis_name="c", subcore_axis_name="s"),
        scratch_types=[
            pltpu.VMEM((nch, K), jnp.int32),
            pltpu.VMEM((K, H), jnp.float32),
            pltpu.VMEM((K, H), jnp.float32),
            pltpu.VMEM((K, H), jnp.float32),
            pltpu.VMEM((K, H), jnp.float32),
            pltpu.SemaphoreType.DMA,
            pltpu.SemaphoreType.DMA,
            pltpu.SemaphoreType.DMA,
            pltpu.SemaphoreType.DMA,
            pltpu.SemaphoreType.DMA,
            pltpu.SemaphoreType.DMA,
            pltpu.SemaphoreType.DMA,
            pltpu.SemaphoreType.DMA,
        ],
    )


_sc_gathers = [_make_sc_gather(nch) for nch in NCHS]


def _mm_compute(g_ref, ids_ref, wt_ref, tt_ref, wb_ref, b_ref, out_ref, small_ref):
    @pl.when(pl.program_id(0) == 0)
    def _():
        small_ref[...] = (
            jnp.dot(tt_ref[...], wb_ref[...], preferred_element_type=jnp.float32)
            + b_ref[...])

    ids = ids_ref[0, 0, :]
    onehot = (ids[:, None] == lax.broadcasted_iota(jnp.int32, (1, VY), 1)
              ).astype(jnp.float32)
    out_ref[...] = (
        jnp.dot(g_ref[...], wt_ref[...], preferred_element_type=jnp.float32)
        + jnp.dot(onehot, small_ref[...], preferred_element_type=jnp.float32))


def _mm_body_first(g_ref, ids_ref, wt_ref, tt_ref, wb_ref, b_ref, out_ref, small_ref):
    _mm_compute(g_ref, ids_ref, wt_ref, tt_ref, wb_ref, b_ref, out_ref, small_ref)


def _mm_body_chained(g_ref, ids_ref, wt_ref, tt_ref, wb_ref, b_ref, prev_ref,
                     out_ref, small_ref):
    del prev_ref
    _mm_compute(g_ref, ids_ref, wt_ref, tt_ref, wb_ref, b_ref, out_ref, small_ref)


def _make_mm(c):
    off = BLK_OFFS[c]
    in_specs = [
        pl.BlockSpec((BN, H), lambda i: (i, 0)),
        pl.BlockSpec((1, 1, BN), lambda i, off=off: (i + off, 0, 0)),
        pl.BlockSpec((H, H), lambda i: (0, 0)),
        pl.BlockSpec((VY, H), lambda i: (0, 0)),
        pl.BlockSpec((H, H), lambda i: (1, 0)),
        pl.BlockSpec((1, H), lambda i: (0, 0)),
    ]
    kwargs = {}
    body = _mm_body_first
    if c > 0:
        in_specs.append(pl.BlockSpec(memory_space=pl.ANY))
        kwargs["input_output_aliases"] = {6: 0}
        body = _mm_body_chained
    return pl.pallas_call(
        body,
        grid=(NBCS[c],),
        in_specs=in_specs,
        out_specs=pl.BlockSpec((BN, H), lambda i, off=off: (i + off, 0)),
        out_shape=jax.ShapeDtypeStruct((N, H), jnp.float32),
        scratch_shapes=[pltpu.VMEM((VY, H), jnp.float32)],
        **kwargs)


_mms = [_make_mm(c) for c in range(C)]


def kernel(token_ids, type_ids, token_table, type_table, W, b):
    tok = jnp.pad(token_ids.astype(jnp.int32), (0, NPAD - N))
    ty = jnp.pad(type_ids.astype(jnp.int32), (0, NPAD - N))
    ids3d = ty.reshape(NB, 1, BN)
    b2 = b.reshape(1, H)

    gs = []
    row = 0
    for c in range(C):
        idx3d = lax.slice(tok, (row,), (row + CHUNKS[c],)).reshape(NW, NCHS[c], K)
        gs.append(_sc_gathers[c](token_table, idx3d))
        row += CHUNKS[c]
    out = _mms[0](gs[0], ids3d, W, type_table, W, b2)
    for c in range(1, C):
        out = _mms[c](gs[c], ids3d, W, type_table, W, b2, out)
    return out


# speedup vs baseline: 1.0065x; 1.0051x over previous
import jax
import jax.numpy as jnp
from jax import lax
from jax.experimental import pallas as pl
from jax.experimental.pallas import tpu as pltpu
from jax.experimental.pallas import tpu_sc as plsc

N = 50000
H = 512
VY = 64

NW = 32
K = 56
NCHS = [2, 10, 8, 8]
C = len(NCHS)
BPWS = [n * K for n in NCHS]
CHUNKS = [bpw * NW for bpw in BPWS]
NPAD = sum(CHUNKS)

BN = 3584
NBCS = [ch // BN for ch in CHUNKS]
BLK_OFFS = [sum(NBCS[:c]) for c in range(C)]
NB = sum(NBCS)


def _make_sc_gather(nch):
    bpw = nch * K

    def body(table_hbm, idx_hbm, out_hbm, idx_v,
             buf0, buf1, buf2, buf3, g0, g1, g2, g3, s0, s1, s2, s3):
        wid = lax.axis_index("s") * 2 + lax.axis_index("c")
        base = wid * bpw
        pltpu.sync_copy(idx_hbm.at[wid], idx_v)
        bufs = (buf0, buf1, buf2, buf3)
        gsems = (g0, g1, g2, g3)
        ssems = (s0, s1, s2, s3)
        gh = [None] * 4
        sh = [None] * 4
        gh[0] = pltpu.async_copy(table_hbm.at[idx_v.at[0]], buf0, g0)
        for j in range(nch):
            p = j % 4
            if j + 1 < nch:
                q = (j + 1) % 4
                if j + 1 >= 4:
                    sh[q].wait()
                gh[q] = pltpu.async_copy(
                    table_hbm.at[idx_v.at[j + 1]], bufs[q], gsems[q])
            gh[p].wait()
            sh[p] = pltpu.async_copy(
                bufs[p], out_hbm.at[pl.ds(base + j * K, K)], ssems[p])
        for p in range(min(nch, 4)):
            sh[(nch - 1 - p) % 4].wait()

    return pl.kernel(
        body,
        out_type=jax.ShapeDtypeStruct((nch * K * NW, H), jnp.float32),
        mesh=plsc.VectorSubcoreMesh(core_axis_name="c", subcore_axis_name="s"),
        scratch_types=[
            pltpu.VMEM((nch, K), jnp.int32),
            pltpu.VMEM((K, H), jnp.float32),
            pltpu.VMEM((K, H), jnp.float32),
            pltpu.VMEM((K, H), jnp.float32),
            pltpu.VMEM((K, H), jnp.float32),
            pltpu.SemaphoreType.DMA,
            pltpu.SemaphoreType.DMA,
            pltpu.SemaphoreType.DMA,
            pltpu.SemaphoreType.DMA,
            pltpu.SemaphoreType.DMA,
            pltpu.SemaphoreType.DMA,
            pltpu.SemaphoreType.DMA,
            pltpu.SemaphoreType.DMA,
        ],
    )


_sc_gathers = [_make_sc_gather(nch) for nch in NCHS]


def _mm_compute(g_ref, ids_ref, wt_ref, tt_ref, wb_ref, b_ref, out_ref, small_ref):
    @pl.when(pl.program_id(0) == 0)
    def _():
        small_ref[...] = (
            jnp.dot(tt_ref[...], wb_ref[...], preferred_element_type=jnp.float32)
            + b_ref[...])

    ids = ids_ref[0, 0, :]
    onehot = (ids[:, None] == lax.broadcasted_iota(jnp.int32, (1, VY), 1)
              ).astype(jnp.float32)
    out_ref[...] = (
        jnp.dot(g_ref[...], wt_ref[...], preferred_element_type=jnp.float32)
        + jnp.dot(onehot, small_ref[...], preferred_element_type=jnp.float32))


def _mm_body_first(g_ref, ids_ref, wt_ref, tt_ref, wb_ref, b_ref, out_ref, small_ref):
    _mm_compute(g_ref, ids_ref, wt_ref, tt_ref, wb_ref, b_ref, out_ref, small_ref)


def _mm_body_chained(g_ref, ids_ref, wt_ref, tt_ref, wb_ref, b_ref, prev_ref,
                     out_ref, small_ref):
    del prev_ref
    _mm_compute(g_ref, ids_ref, wt_ref, tt_ref, wb_ref, b_ref, out_ref, small_ref)


def _make_mm(c):
    off = BLK_OFFS[c]
    in_specs = [
        pl.BlockSpec((BN, H), lambda i: (i, 0)),
        pl.BlockSpec((1, 1, BN), lambda i, off=off: (i + off, 0, 0)),
        pl.BlockSpec((H, H), lambda i: (0, 0)),
        pl.BlockSpec((VY, H), lambda i: (0, 0)),
        pl.BlockSpec((H, H), lambda i: (1, 0)),
        pl.BlockSpec((1, H), lambda i: (0, 0)),
    ]
    kwargs = {}
    body = _mm_body_first
    if c > 0:
        in_specs.append(pl.BlockSpec(memory_space=pl.ANY))
        kwargs["input_output_aliases"] = {6: 0}
        body = _mm_body_chained
    return pl.pallas_call(
        body,
        grid=(NBCS[c],),
        in_specs=in_specs,
        out_specs=pl.BlockSpec((BN, H), lambda i, off=off: (i + off, 0)),
        out_shape=jax.ShapeDtypeStruct((N, H), jnp.float32),
        scratch_shapes=[pltpu.VMEM((VY, H), jnp.float32)],
        **kwargs)


_mms = [_make_mm(c) for c in range(C)]


def kernel(token_ids, type_ids, token_table, type_table, W, b):
    tok = jnp.pad(token_ids.astype(jnp.int32), (0, NPAD - N))
    ty = jnp.pad(type_ids.astype(jnp.int32), (0, NPAD - N))
    ids3d = ty.reshape(NB, 1, BN)
    b2 = b.reshape(1, H)

    gs = []
    row = 0
    for c in range(C):
        idx3d = lax.slice(tok, (row,), (row + CHUNKS[c],)).reshape(NW, NCHS[c], K)
        gs.append(_sc_gathers[c](token_table, idx3d))
        row += CHUNKS[c]
    out = _mms[0](gs[0], ids3d, W, type_table, W, b2)
    for c in range(1, C):
        out = _mms[c](gs[c], ids3d, W, type_table, W, b2, out)
    return out
